# Initial kernel scaffold; baseline (speedup 1.0000x reference)
import jax
import jax.numpy as jnp
from jax.experimental import pallas as pl


def kernel(x, y, memory):
    raise NotImplementedError("write your pallas kernel here")


# speedup vs baseline: 8.6734x; 8.6734x over previous
import functools

import numpy as np
import jax
import jax.numpy as jnp
from jax import lax
from jax.experimental import pallas as pl
from jax.experimental.pallas import tpu as pltpu
from jax.experimental.pallas import tpu_sc as plsc

N_ROWS = 100000
N_DIM = 256
KP1 = 2049
BSZ = 128
INV_T = 1.0 / 0.07

BM = 2000
GRID_M = N_ROWS // BM


def _matmul_body(mem_ref, x_ref, out_ref):
    xs = x_ref[...] * INV_T
    out_ref[...] = lax.dot_general(
        mem_ref[...], xs,
        dimension_numbers=(((1,), (1,)), ((), ())),
        preferred_element_type=jnp.float32,
        precision=lax.Precision.HIGHEST,
    )


def _matmul(memory, x):
    return pl.pallas_call(
        _matmul_body,
        grid=(GRID_M,),
        in_specs=[
            pl.BlockSpec((BM, N_DIM), lambda i: (i, 0)),
            pl.BlockSpec((BSZ, N_DIM), lambda i: (0, 0)),
        ],
        out_specs=pl.BlockSpec((BM, BSZ), lambda i: (i, 0)),
        out_shape=jax.ShapeDtypeStruct((N_ROWS, BSZ), jnp.float32),
    )(memory, x)


NW = 32
CHUNK = 128
PER_W_CHUNKS = 65
N_CHUNKS = NW * PER_W_CHUNKS
N_PAD = N_CHUNKS * CHUNK
UNROLL = 5

PER_W = PER_W_CHUNKS * CHUNK


@functools.cache
def _gather_kernel():
    mesh = plsc.VectorSubcoreMesh(core_axis_name="c", subcore_axis_name="s")

    @functools.partial(
        pl.kernel,
        mesh=mesh,
        out_type=jax.ShapeDtypeStruct((N_PAD,), jnp.float32),
        scratch_types=[
            pltpu.VMEM((PER_W,), jnp.int32),
            pltpu.VMEM((PER_W,), jnp.float32),
            pltpu.SemaphoreType.DMA,
        ],
    )
    def gather(p_hbm, idx_hbm, out_hbm, idx_v, val_v, sem):
        wid = lax.axis_index("s") * 2 + lax.axis_index("c")
        base = wid * PER_W
        pltpu.sync_copy(idx_hbm.at[pl.ds(base, PER_W)], idx_v)

        def body(g, carry):
            handles = []
            for u in range(UNROLL):
                off = (g * UNROLL + u) * CHUNK
                handles.append(
                    pltpu.async_copy(p_hbm.at[idx_v.at[pl.ds(off, CHUNK)]],
                                     val_v.at[pl.ds(off, CHUNK)], sem))
            for h in handles:
                h.wait()
            return carry

        lax.fori_loop(0, PER_W_CHUNKS // UNROLL, body, 0)
        pltpu.sync_copy(val_v, out_hbm.at[pl.ds(base, PER_W)])

    return gather


def _threefry2x32(kpair, x0, x1):
    rot = ((13, 15, 26, 6), (17, 29, 16, 24))

    def rotl(v, d):
        return ((v << np.uint32(d)) | (v >> np.uint32(32 - d))).astype(np.uint32)

    ks = (np.uint32(kpair[0]), np.uint32(kpair[1]),
          np.uint32(kpair[0] ^ kpair[1] ^ np.uint32(0x1BD11BDA)))
    with np.errstate(over="ignore"):
        a = (x0 + ks[0]).astype(np.uint32)
        b = (x1 + ks[1]).astype(np.uint32)
        for i in range(5):
            for r in rot[i % 2]:
                a = (a + b).astype(np.uint32)
                b = rotl(b, r) ^ a
            a = (a + ks[(i + 1) % 3]).astype(np.uint32)
            b = (b + ks[(i + 2) % 3] + np.uint32(i + 1)).astype(np.uint32)
    return a, b


def _threefry_bits(kpair, size):
    a, b = _threefry2x32(kpair, np.zeros(size, np.uint32),
                         np.arange(size, dtype=np.uint32))
    return a ^ b


def _randint_key1(shape, span):
    a, b = _threefry2x32((np.uint32(0), np.uint32(1)),
                         np.zeros(2, np.uint32),
                         np.arange(2, dtype=np.uint32))
    key_hi = (a[0], b[0])
    key_lo = (a[1], b[1])
    n = int(np.prod(shape))
    hi = _threefry_bits(key_hi, n)
    lo = _threefry_bits(key_lo, n)
    s = np.uint32(span)
    with np.errstate(over="ignore"):
        mult = np.uint32(65536) % s
        mult = np.uint32(mult * mult) % s
        vals = (np.uint32(np.uint32(hi % s) * mult) + (lo % s)) % s
    return vals.astype(np.int64).reshape(shape)


def _flat_idx_base():
    idx = _randint_key1((BSZ, KP1), N_ROWS)
    b = np.arange(BSZ, dtype=np.int64)[:, None]
    flat = (idx * BSZ + b).reshape(-1)
    out = np.zeros((N_PAD,), dtype=np.int32)
    out[: flat.size] = flat.astype(np.int32)
    return out


_FLAT_IDX_BASE = _flat_idx_base()


def kernel(x, y, memory):
    p = _matmul(memory, x)
    p_flat = p.reshape(N_ROWS * BSZ)

    flat_idx = jnp.asarray(_FLAT_IDX_BASE)
    col0_pos = jnp.arange(BSZ, dtype=jnp.int32) * KP1
    col0_val = y.astype(jnp.int32) * BSZ + jnp.arange(BSZ, dtype=jnp.int32)
    flat_idx = flat_idx.at[col0_pos].set(col0_val)

    out = _gather_kernel()(p_flat, flat_idx)
    logits = out.reshape(N_PAD)[: BSZ * KP1].reshape(BSZ, KP1)
    labels = jnp.zeros((BSZ,), dtype=jnp.int32)
    return (logits, labels)


# speedup vs baseline: 10.9833x; 1.2663x over previous
import functools

import numpy as np
import jax
import jax.numpy as jnp
from jax import lax
from jax.experimental import pallas as pl
from jax.experimental.pallas import tpu as pltpu
from jax.experimental.pallas import tpu_sc as plsc

N_ROWS = 100000
N_DIM = 256
KP1 = 2049
BSZ = 128
INV_T = 1.0 / 0.07

BM = 2000
GRID_M = N_ROWS // BM


def _matmul_body(mem_ref, x_ref, out_ref):
    xs = x_ref[...] * INV_T
    out_ref[...] = lax.dot_general(
        mem_ref[...], xs,
        dimension_numbers=(((1,), (1,)), ((), ())),
        preferred_element_type=jnp.float32,
        precision=lax.Precision.DEFAULT,
    )


def _matmul(memory, x):
    return pl.pallas_call(
        _matmul_body,
        grid=(GRID_M,),
        in_specs=[
            pl.BlockSpec((BM, N_DIM), lambda i: (i, 0)),
            pl.BlockSpec((BSZ, N_DIM), lambda i: (0, 0)),
        ],
        out_specs=pl.BlockSpec((BM, BSZ), lambda i: (i, 0)),
        out_shape=jax.ShapeDtypeStruct((N_ROWS, BSZ), jnp.float32),
    )(memory, x)


NW = 32
CHUNK = 128
PER_W_CHUNKS = 65
N_CHUNKS = NW * PER_W_CHUNKS
N_PAD = N_CHUNKS * CHUNK
UNROLL = 5

PER_W = PER_W_CHUNKS * CHUNK


@functools.cache
def _gather_kernel():
    mesh = plsc.VectorSubcoreMesh(core_axis_name="c", subcore_axis_name="s")

    @functools.partial(
        pl.kernel,
        mesh=mesh,
        out_type=jax.ShapeDtypeStruct((N_PAD,), jnp.float32),
        scratch_types=[
            pltpu.VMEM((PER_W,), jnp.int32),
            pltpu.VMEM((PER_W,), jnp.float32),
            pltpu.SemaphoreType.DMA,
        ],
    )
    def gather(p_hbm, idx_hbm, out_hbm, idx_v, val_v, sem):
        wid = lax.axis_index("s") * 2 + lax.axis_index("c")
        base = wid * PER_W
        pltpu.sync_copy(idx_hbm.at[pl.ds(base, PER_W)], idx_v)

        def body(g, carry):
            handles = []
            for u in range(UNROLL):
                off = (g * UNROLL + u) * CHUNK
                handles.append(
                    pltpu.async_copy(p_hbm.at[idx_v.at[pl.ds(off, CHUNK)]],
                                     val_v.at[pl.ds(off, CHUNK)], sem))
            for h in handles:
                h.wait()
            return carry

        lax.fori_loop(0, PER_W_CHUNKS // UNROLL, body, 0)
        pltpu.sync_copy(val_v, out_hbm.at[pl.ds(base, PER_W)])

    return gather


def _threefry2x32(kpair, x0, x1):
    rot = ((13, 15, 26, 6), (17, 29, 16, 24))

    def rotl(v, d):
        return ((v << np.uint32(d)) | (v >> np.uint32(32 - d))).astype(np.uint32)

    ks = (np.uint32(kpair[0]), np.uint32(kpair[1]),
          np.uint32(kpair[0] ^ kpair[1] ^ np.uint32(0x1BD11BDA)))
    with np.errstate(over="ignore"):
        a = (x0 + ks[0]).astype(np.uint32)
        b = (x1 + ks[1]).astype(np.uint32)
        for i in range(5):
            for r in rot[i % 2]:
                a = (a + b).astype(np.uint32)
                b = rotl(b, r) ^ a
            a = (a + ks[(i + 1) % 3]).astype(np.uint32)
            b = (b + ks[(i + 2) % 3] + np.uint32(i + 1)).astype(np.uint32)
    return a, b


def _threefry_bits(kpair, size):
    a, b = _threefry2x32(kpair, np.zeros(size, np.uint32),
                         np.arange(size, dtype=np.uint32))
    return a ^ b


def _randint_key1(shape, span):
    a, b = _threefry2x32((np.uint32(0), np.uint32(1)),
                         np.zeros(2, np.uint32),
                         np.arange(2, dtype=np.uint32))
    key_hi = (a[0], b[0])
    key_lo = (a[1], b[1])
    n = int(np.prod(shape))
    hi = _threefry_bits(key_hi, n)
    lo = _threefry_bits(key_lo, n)
    s = np.uint32(span)
    with np.errstate(over="ignore"):
        mult = np.uint32(65536) % s
        mult = np.uint32(mult * mult) % s
        vals = (np.uint32(np.uint32(hi % s) * mult) + (lo % s)) % s
    return vals.astype(np.int64).reshape(shape)


def _flat_idx_base():
    idx = _randint_key1((BSZ, KP1), N_ROWS)
    b = np.arange(BSZ, dtype=np.int64)[:, None]
    flat = (idx * BSZ + b).reshape(-1)
    out = np.zeros((N_PAD,), dtype=np.int32)
    out[: flat.size] = flat.astype(np.int32)
    return out


_FLAT_IDX_BASE = _flat_idx_base()


def kernel(x, y, memory):
    p = _matmul(memory, x)
    p_flat = p.reshape(N_ROWS * BSZ)

    flat_idx = jnp.asarray(_FLAT_IDX_BASE)
    col0_pos = jnp.arange(BSZ, dtype=jnp.int32) * KP1
    col0_val = y.astype(jnp.int32) * BSZ + jnp.arange(BSZ, dtype=jnp.int32)
    flat_idx = flat_idx.at[col0_pos].set(col0_val)

    out = _gather_kernel()(p_flat, flat_idx)
    logits = out.reshape(N_PAD)[: BSZ * KP1].reshape(BSZ, KP1)
    labels = jnp.zeros((BSZ,), dtype=jnp.int32)
    return (logits, labels)


# speedup vs baseline: 13.4901x; 1.2282x over previous
import functools

import numpy as np
import jax
import jax.numpy as jnp
from jax import lax
from jax.experimental import pallas as pl
from jax.experimental.pallas import tpu as pltpu
from jax.experimental.pallas import tpu_sc as plsc

N_ROWS = 100000
N_DIM = 256
KP1 = 2049
BSZ = 128
INV_T = 1.0 / 0.07

BM = 2000
GRID_M = N_ROWS // BM


def _matmul_body(mem_ref, x_ref, out_ref):
    xs = x_ref[...] * INV_T
    out_ref[...] = lax.dot_general(
        mem_ref[...], xs,
        dimension_numbers=(((1,), (1,)), ((), ())),
        preferred_element_type=jnp.float32,
        precision=lax.Precision.DEFAULT,
    )


def _matmul(memory, x):
    return pl.pallas_call(
        _matmul_body,
        grid=(GRID_M,),
        in_specs=[
            pl.BlockSpec((BM, N_DIM), lambda i: (i, 0)),
            pl.BlockSpec((BSZ, N_DIM), lambda i: (0, 0)),
        ],
        out_specs=pl.BlockSpec((BM, BSZ), lambda i: (i, 0)),
        out_shape=jax.ShapeDtypeStruct((N_ROWS, BSZ), jnp.float32),
    )(memory, x)


NW = 32
CHUNK = 128
W0_CHUNKS = 65
W_CHUNKS = 64
W0_N = W0_CHUNKS * CHUNK
W_N = W_CHUNKS * CHUNK
N_OUT = BSZ * KP1
IDX_PAD = W0_N + (NW - 1) * W_N + (W0_N - W_N)
DEPTH = 12


@functools.cache
def _gather_kernel():
    mesh = plsc.VectorSubcoreMesh(core_axis_name="c", subcore_axis_name="s")

    @functools.partial(
        pl.kernel,
        mesh=mesh,
        out_type=jax.ShapeDtypeStruct((N_OUT,), jnp.float32),
        scratch_types=[
            pltpu.VMEM((W0_N,), jnp.int32),
            pltpu.VMEM((W0_N,), jnp.float32),
            pltpu.SemaphoreType.DMA,
        ],
    )
    def gather(p_hbm, idx_hbm, ycol_hbm, out_hbm, idx_v, val_v, sem):
        wid = lax.axis_index("s") * 2 + lax.axis_index("c")
        base = jnp.where(wid == 0, 0, W0_N + (wid - 1) * W_N)
        nch = jnp.where(wid == 0, W0_CHUNKS, W_CHUNKS)
        pltpu.sync_copy(idx_hbm.at[pl.ds(pl.multiple_of(base, 8), W0_N)],
                        idx_v)

        @pl.when(wid == 0)
        def _():
            pltpu.sync_copy(ycol_hbm, idx_v.at[pl.ds(0, BSZ)])

        def issue(g):
            off = pl.multiple_of(g * CHUNK, CHUNK)
            return pltpu.async_copy(
                p_hbm.at[idx_v.at[pl.ds(off, CHUNK)]],
                val_v.at[pl.ds(off, CHUNK)], sem)

        def wait_one():
            pltpu.make_async_copy(p_hbm.at[pl.ds(0, CHUNK)],
                                  val_v.at[pl.ds(0, CHUNK)], sem).wait()

        for g in range(DEPTH):
            issue(g)

        def body(g, carry):
            issue(g + DEPTH)
            wait_one()
            return carry

        lax.fori_loop(0, nch - DEPTH, body, 0)
        for _ in range(DEPTH):
            wait_one()

        @pl.when(wid == 0)
        def _():
            pltpu.sync_copy(val_v, out_hbm.at[pl.ds(0, W0_N)])

        @pl.when(wid > 0)
        def _():
            pltpu.sync_copy(
                val_v.at[pl.ds(0, W_N)],
                out_hbm.at[pl.ds(pl.multiple_of(base, 8), W_N)])

    return gather


def _threefry2x32(kpair, x0, x1):
    rot = ((13, 15, 26, 6), (17, 29, 16, 24))

    def rotl(v, d):
        return ((v << np.uint32(d)) | (v >> np.uint32(32 - d))).astype(np.uint32)

    ks = (np.uint32(kpair[0]), np.uint32(kpair[1]),
          np.uint32(kpair[0] ^ kpair[1] ^ np.uint32(0x1BD11BDA)))
    with np.errstate(over="ignore"):
        a = (x0 + ks[0]).astype(np.uint32)
        b = (x1 + ks[1]).astype(np.uint32)
        for i in range(5):
            for r in rot[i % 2]:
                a = (a + b).astype(np.uint32)
                b = rotl(b, r) ^ a
            a = (a + ks[(i + 1) % 3]).astype(np.uint32)
            b = (b + ks[(i + 2) % 3] + np.uint32(i + 1)).astype(np.uint32)
    return a, b


def _threefry_bits(kpair, size):
    a, b = _threefry2x32(kpair, np.zeros(size, np.uint32),
                         np.arange(size, dtype=np.uint32))
    return a ^ b


def _randint_key1(shape, span):
    a, b = _threefry2x32((np.uint32(0), np.uint32(1)),
                         np.zeros(2, np.uint32),
                         np.arange(2, dtype=np.uint32))
    key_hi = (a[0], b[0])
    key_lo = (a[1], b[1])
    n = int(np.prod(shape))
    hi = _threefry_bits(key_hi, n)
    lo = _threefry_bits(key_lo, n)
    s = np.uint32(span)
    with np.errstate(over="ignore"):
        mult = np.uint32(65536) % s
        mult = np.uint32(mult * mult) % s
        vals = (np.uint32(np.uint32(hi % s) * mult) + (lo % s)) % s
    return vals.astype(np.int64).reshape(shape)


def _flat_idx_base():
    idx = _randint_key1((BSZ, KP1), N_ROWS)
    b = np.arange(BSZ, dtype=np.int64)[None, :]
    flat = (idx.T * BSZ + b).reshape(-1)
    out = np.zeros((IDX_PAD,), dtype=np.int32)
    out[: flat.size] = flat.astype(np.int32)
    return out


_FLAT_IDX_BASE = _flat_idx_base()


def kernel(x, y, memory):
    p = _matmul(memory, x)
    p_flat = p.reshape(N_ROWS * BSZ)
    flat_idx = jnp.asarray(_FLAT_IDX_BASE)
    ycol = y.astype(jnp.int32) * BSZ + jnp.arange(BSZ, dtype=jnp.int32)
    out = _gather_kernel()(p_flat, flat_idx, ycol)
    logits = out.reshape(KP1, BSZ).T
    labels = jnp.zeros((BSZ,), dtype=jnp.int32)
    return (logits, labels)


# speedup vs baseline: 16.3453x; 1.2117x over previous
import functools

import numpy as np
import jax
import jax.numpy as jnp
from jax import lax
from jax.experimental import pallas as pl
from jax.experimental.pallas import tpu as pltpu
from jax.experimental.pallas import tpu_sc as plsc

N_ROWS = 100000
N_DIM = 256
KP1 = 2049
BSZ = 128
INV_T = 1.0 / 0.07

BM = 10000
GRID_M = N_ROWS // BM


def _matmul_body(mem_ref, x_ref, out_ref):
    xs = x_ref[...] * INV_T
    out_ref[...] = lax.dot_general(
        mem_ref[...], xs,
        dimension_numbers=(((1,), (1,)), ((), ())),
        preferred_element_type=jnp.float32,
        precision=lax.Precision.DEFAULT,
    )


def _matmul(memory, x):
    return pl.pallas_call(
        _matmul_body,
        grid=(GRID_M,),
        in_specs=[
            pl.BlockSpec((BM, N_DIM), lambda i: (i, 0)),
            pl.BlockSpec((BSZ, N_DIM), lambda i: (0, 0)),
        ],
        out_specs=pl.BlockSpec((BM, BSZ), lambda i: (i, 0)),
        out_shape=jax.ShapeDtypeStruct((N_ROWS, BSZ), jnp.float32),
    )(memory, x)


NW = 32
CHUNK = 128
W0_CHUNKS = 65
W_CHUNKS = 64
W0_N = W0_CHUNKS * CHUNK
W_N = W_CHUNKS * CHUNK
N_OUT = BSZ * KP1
IDX_PAD = W0_N + (NW - 1) * W_N + (W0_N - W_N)
DEPTH = 12


@functools.cache
def _gather_kernel():
    mesh = plsc.VectorSubcoreMesh(core_axis_name="c", subcore_axis_name="s")

    @functools.partial(
        pl.kernel,
        mesh=mesh,
        out_type=jax.ShapeDtypeStruct((N_OUT,), jnp.float32),
        scratch_types=[
            pltpu.VMEM((W0_N,), jnp.int32),
            pltpu.VMEM((W0_N,), jnp.float32),
            pltpu.SemaphoreType.DMA,
        ],
    )
    def gather(p_hbm, idx_hbm, ycol_hbm, out_hbm, idx_v, val_v, sem):
        wid = lax.axis_index("s") * 2 + lax.axis_index("c")
        base = jnp.where(wid == 0, 0, W0_N + (wid - 1) * W_N)
        nch = jnp.where(wid == 0, W0_CHUNKS, W_CHUNKS)
        pltpu.sync_copy(idx_hbm.at[pl.ds(pl.multiple_of(base, 8), W0_N)],
                        idx_v)

        @pl.when(wid == 0)
        def _():
            pltpu.sync_copy(ycol_hbm, idx_v.at[pl.ds(0, BSZ)])

        def issue(g):
            off = pl.multiple_of(g * CHUNK, CHUNK)
            return pltpu.async_copy(
                p_hbm.at[idx_v.at[pl.ds(off, CHUNK)]],
                val_v.at[pl.ds(off, CHUNK)], sem)

        def wait_one():
            pltpu.make_async_copy(p_hbm.at[pl.ds(0, CHUNK)],
                                  val_v.at[pl.ds(0, CHUNK)], sem).wait()

        for g in range(DEPTH):
            issue(g)

        def body(g, carry):
            issue(g + DEPTH)
            wait_one()
            return carry

        lax.fori_loop(0, nch - DEPTH, body, 0)
        for _ in range(DEPTH):
            wait_one()

        @pl.when(wid == 0)
        def _():
            pltpu.sync_copy(val_v, out_hbm.at[pl.ds(0, W0_N)])

        @pl.when(wid > 0)
        def _():
            pltpu.sync_copy(
                val_v.at[pl.ds(0, W_N)],
                out_hbm.at[pl.ds(pl.multiple_of(base, 8), W_N)])

    return gather


def _threefry2x32(kpair, x0, x1):
    rot = ((13, 15, 26, 6), (17, 29, 16, 24))

    def rotl(v, d):
        return ((v << np.uint32(d)) | (v >> np.uint32(32 - d))).astype(np.uint32)

    ks = (np.uint32(kpair[0]), np.uint32(kpair[1]),
          np.uint32(kpair[0] ^ kpair[1] ^ np.uint32(0x1BD11BDA)))
    with np.errstate(over="ignore"):
        a = (x0 + ks[0]).astype(np.uint32)
        b = (x1 + ks[1]).astype(np.uint32)
        for i in range(5):
            for r in rot[i % 2]:
                a = (a + b).astype(np.uint32)
                b = rotl(b, r) ^ a
            a = (a + ks[(i + 1) % 3]).astype(np.uint32)
            b = (b + ks[(i + 2) % 3] + np.uint32(i + 1)).astype(np.uint32)
    return a, b


def _threefry_bits(kpair, size):
    a, b = _threefry2x32(kpair, np.zeros(size, np.uint32),
                         np.arange(size, dtype=np.uint32))
    return a ^ b


def _randint_key1(shape, span):
    a, b = _threefry2x32((np.uint32(0), np.uint32(1)),
                         np.zeros(2, np.uint32),
                         np.arange(2, dtype=np.uint32))
    key_hi = (a[0], b[0])
    key_lo = (a[1], b[1])
    n = int(np.prod(shape))
    hi = _threefry_bits(key_hi, n)
    lo = _threefry_bits(key_lo, n)
    s = np.uint32(span)
    with np.errstate(over="ignore"):
        mult = np.uint32(65536) % s
        mult = np.uint32(mult * mult) % s
        vals = (np.uint32(np.uint32(hi % s) * mult) + (lo % s)) % s
    return vals.astype(np.int64).reshape(shape)


def _flat_idx_base():
    idx = _randint_key1((BSZ, KP1), N_ROWS)
    b = np.arange(BSZ, dtype=np.int64)[None, :]
    flat = (idx.T * BSZ + b).reshape(-1)
    out = np.zeros((IDX_PAD,), dtype=np.int32)
    out[: flat.size] = flat.astype(np.int32)
    return out


_FLAT_IDX_BASE = _flat_idx_base()


def kernel(x, y, memory):
    p = _matmul(memory, x)
    p_flat = p.reshape(N_ROWS * BSZ)
    flat_idx = jnp.asarray(_FLAT_IDX_BASE)
    ycol = y.astype(jnp.int32) * BSZ + jnp.arange(BSZ, dtype=jnp.int32)
    out = _gather_kernel()(p_flat, flat_idx, ycol)
    logits = out.reshape(KP1, BSZ).T
    labels = jnp.zeros((BSZ,), dtype=jnp.int32)
    return (logits, labels)


# speedup vs baseline: 16.4120x; 1.0041x over previous
import functools

import numpy as np
import jax
import jax.numpy as jnp
from jax import lax
from jax.experimental import pallas as pl
from jax.experimental.pallas import tpu as pltpu
from jax.experimental.pallas import tpu_sc as plsc

N_ROWS = 100000
N_DIM = 256
KP1 = 2049
BSZ = 128
INV_T = 1.0 / 0.07

BM = 10000
GRID_M = N_ROWS // BM


def _matmul_body(mem_ref, x_ref, out_ref):
    xs = x_ref[...] * INV_T
    out_ref[...] = lax.dot_general(
        mem_ref[...], xs,
        dimension_numbers=(((1,), (1,)), ((), ())),
        preferred_element_type=jnp.float32,
        precision=lax.Precision.DEFAULT,
    )


def _matmul(memory, x):
    return pl.pallas_call(
        _matmul_body,
        grid=(GRID_M,),
        in_specs=[
            pl.BlockSpec((BM, N_DIM), lambda i: (i, 0)),
            pl.BlockSpec((BSZ, N_DIM), lambda i: (0, 0)),
        ],
        out_specs=pl.BlockSpec((BM, BSZ), lambda i: (i, 0)),
        out_shape=jax.ShapeDtypeStruct((N_ROWS, BSZ), jnp.float32),
    )(memory, x)


NW = 32
CHUNK = 128
W0_CHUNKS = 65
W_CHUNKS = 64
W0_N = W0_CHUNKS * CHUNK
W_N = W_CHUNKS * CHUNK
N_OUT = BSZ * KP1
IDX_PAD = W0_N + (NW - 1) * W_N + (W0_N - W_N)
DEPTH = 12


@functools.cache
def _gather_kernel():
    mesh = plsc.VectorSubcoreMesh(core_axis_name="c", subcore_axis_name="s")

    @functools.partial(
        pl.kernel,
        mesh=mesh,
        out_type=jax.ShapeDtypeStruct((N_OUT,), jnp.float32),
        scratch_types=[
            pltpu.VMEM((W0_N,), jnp.int32),
            pltpu.VMEM((W0_N,), jnp.float32),
            pltpu.SemaphoreType.DMA,
        ],
    )
    def gather(p_hbm, idx_hbm, ycol_hbm, out_hbm, idx_v, val_v, sem):
        wid = lax.axis_index("s") * 2 + lax.axis_index("c")
        base = jnp.where(wid == 0, 0, W0_N + (wid - 1) * W_N)
        nch = jnp.where(wid == 0, W0_CHUNKS, W_CHUNKS)
        pltpu.sync_copy(idx_hbm.at[pl.ds(pl.multiple_of(base, 8), W0_N)],
                        idx_v)

        @pl.when(wid == 0)
        def _():
            pltpu.sync_copy(ycol_hbm, idx_v.at[pl.ds(0, BSZ)])

        def issue(g):
            off = pl.multiple_of(g * CHUNK, CHUNK)
            return pltpu.async_copy(
                p_hbm.at[idx_v.at[pl.ds(off, CHUNK)]],
                val_v.at[pl.ds(off, CHUNK)], sem)

        def wait_one():
            pltpu.make_async_copy(p_hbm.at[pl.ds(0, CHUNK)],
                                  val_v.at[pl.ds(0, CHUNK)], sem).wait()

        def pro(g, carry):
            issue(g)
            return carry

        lax.fori_loop(0, DEPTH, pro, 0)

        def body(g, carry):
            issue(g + DEPTH)
            wait_one()
            return carry

        lax.fori_loop(0, nch - DEPTH, body, 0)

        def epi(g, carry):
            wait_one()
            return carry

        lax.fori_loop(0, DEPTH, epi, 0)

        @pl.when(wid == 0)
        def _():
            pltpu.sync_copy(val_v, out_hbm.at[pl.ds(0, W0_N)])

        @pl.when(wid > 0)
        def _():
            pltpu.sync_copy(
                val_v.at[pl.ds(0, W_N)],
                out_hbm.at[pl.ds(pl.multiple_of(base, 8), W_N)])

    return gather


def _threefry2x32(kpair, x0, x1):
    rot = ((13, 15, 26, 6), (17, 29, 16, 24))

    def rotl(v, d):
        return ((v << np.uint32(d)) | (v >> np.uint32(32 - d))).astype(np.uint32)

    ks = (np.uint32(kpair[0]), np.uint32(kpair[1]),
          np.uint32(kpair[0] ^ kpair[1] ^ np.uint32(0x1BD11BDA)))
    with np.errstate(over="ignore"):
        a = (x0 + ks[0]).astype(np.uint32)
        b = (x1 + ks[1]).astype(np.uint32)
        for i in range(5):
            for r in rot[i % 2]:
                a = (a + b).astype(np.uint32)
                b = rotl(b, r) ^ a
            a = (a + ks[(i + 1) % 3]).astype(np.uint32)
            b = (b + ks[(i + 2) % 3] + np.uint32(i + 1)).astype(np.uint32)
    return a, b


def _threefry_bits(kpair, size):
    a, b = _threefry2x32(kpair, np.zeros(size, np.uint32),
                         np.arange(size, dtype=np.uint32))
    return a ^ b


def _randint_key1(shape, span):
    a, b = _threefry2x32((np.uint32(0), np.uint32(1)),
                         np.zeros(2, np.uint32),
                         np.arange(2, dtype=np.uint32))
    key_hi = (a[0], b[0])
    key_lo = (a[1], b[1])
    n = int(np.prod(shape))
    hi = _threefry_bits(key_hi, n)
    lo = _threefry_bits(key_lo, n)
    s = np.uint32(span)
    with np.errstate(over="ignore"):
        mult = np.uint32(65536) % s
        mult = np.uint32(mult * mult) % s
        vals = (np.uint32(np.uint32(hi % s) * mult) + (lo % s)) % s
    return vals.astype(np.int64).reshape(shape)


def _flat_idx_base():
    idx = _randint_key1((BSZ, KP1), N_ROWS)
    b = np.arange(BSZ, dtype=np.int64)[None, :]
    flat = (idx.T * BSZ + b).reshape(-1)
    out = np.zeros((IDX_PAD,), dtype=np.int32)
    out[: flat.size] = flat.astype(np.int32)
    return out


_FLAT_IDX_BASE = _flat_idx_base()


def kernel(x, y, memory):
    p = _matmul(memory, x)
    p_flat = p.reshape(N_ROWS * BSZ)
    flat_idx = jnp.asarray(_FLAT_IDX_BASE)
    ycol = y.astype(jnp.int32) * BSZ + jnp.arange(BSZ, dtype=jnp.int32)
    out = _gather_kernel()(p_flat, flat_idx, ycol)
    logits = out.reshape(KP1, BSZ).T
    labels = jnp.zeros((BSZ,), dtype=jnp.int32)
    return (logits, labels)


# speedup vs baseline: 16.7483x; 1.0205x over previous
import functools

import numpy as np
import jax
import jax.numpy as jnp
from jax import lax
from jax.experimental import pallas as pl
from jax.experimental.pallas import tpu as pltpu
from jax.experimental.pallas import tpu_sc as plsc

N_ROWS = 100000
N_DIM = 256
KP1 = 2049
BSZ = 128
INV_T = 1.0 / 0.07

BM = 10000
GRID_M = N_ROWS // BM


def _matmul_body(mem_ref, x_ref, out_ref):
    xs = x_ref[...] * INV_T
    out_ref[...] = lax.dot_general(
        mem_ref[...], xs,
        dimension_numbers=(((1,), (1,)), ((), ())),
        preferred_element_type=jnp.float32,
        precision=lax.Precision.DEFAULT,
    )


def _matmul(memory, x):
    return pl.pallas_call(
        _matmul_body,
        grid=(GRID_M,),
        in_specs=[
            pl.BlockSpec((BM, N_DIM), lambda i: (i, 0)),
            pl.BlockSpec((BSZ, N_DIM), lambda i: (0, 0)),
        ],
        out_specs=pl.BlockSpec((BM, BSZ), lambda i: (i, 0)),
        out_shape=jax.ShapeDtypeStruct((N_ROWS, BSZ), jnp.float32),
    )(memory, x)


NW = 32
CHUNK = 128
W0_CHUNKS = 65
W_CHUNKS = 64
W0_N = W0_CHUNKS * CHUNK
W_N = W_CHUNKS * CHUNK
N_OUT = BSZ * KP1
IDX_PAD = W0_N + (NW - 1) * W_N + (W0_N - W_N)
DEPTH = 32


@functools.cache
def _gather_kernel():
    mesh = plsc.VectorSubcoreMesh(core_axis_name="c", subcore_axis_name="s")

    @functools.partial(
        pl.kernel,
        mesh=mesh,
        out_type=jax.ShapeDtypeStruct((N_OUT,), jnp.float32),
        scratch_types=[
            pltpu.VMEM((W0_N,), jnp.int32),
            pltpu.VMEM((W0_N,), jnp.float32),
            pltpu.SemaphoreType.DMA,
        ],
    )
    def gather(p_hbm, idx_hbm, ycol_hbm, out_hbm, idx_v, val_v, sem):
        wid = lax.axis_index("s") * 2 + lax.axis_index("c")
        base = jnp.where(wid == 0, 0, W0_N + (wid - 1) * W_N)
        nch = jnp.where(wid == 0, W0_CHUNKS, W_CHUNKS)
        pltpu.sync_copy(idx_hbm.at[pl.ds(pl.multiple_of(base, 8), W0_N)],
                        idx_v)

        @pl.when(wid == 0)
        def _():
            pltpu.sync_copy(ycol_hbm, idx_v.at[pl.ds(0, BSZ)])

        def issue(g):
            off = pl.multiple_of(g * CHUNK, CHUNK)
            return pltpu.async_copy(
                p_hbm.at[idx_v.at[pl.ds(off, CHUNK)]],
                val_v.at[pl.ds(off, CHUNK)], sem)

        def wait_one():
            pltpu.make_async_copy(p_hbm.at[pl.ds(0, CHUNK)],
                                  val_v.at[pl.ds(0, CHUNK)], sem).wait()

        def pro(g, carry):
            issue(g)
            return carry

        lax.fori_loop(0, DEPTH, pro, 0)

        def body(g, carry):
            issue(g + DEPTH)
            wait_one()
            return carry

        lax.fori_loop(0, nch - DEPTH, body, 0)

        def epi(g, carry):
            wait_one()
            return carry

        lax.fori_loop(0, DEPTH, epi, 0)

        @pl.when(wid == 0)
        def _():
            pltpu.sync_copy(val_v, out_hbm.at[pl.ds(0, W0_N)])

        @pl.when(wid > 0)
        def _():
            pltpu.sync_copy(
                val_v.at[pl.ds(0, W_N)],
                out_hbm.at[pl.ds(pl.multiple_of(base, 8), W_N)])

    return gather


def _threefry2x32(kpair, x0, x1):
    rot = ((13, 15, 26, 6), (17, 29, 16, 24))

    def rotl(v, d):
        return ((v << np.uint32(d)) | (v >> np.uint32(32 - d))).astype(np.uint32)

    ks = (np.uint32(kpair[0]), np.uint32(kpair[1]),
          np.uint32(kpair[0] ^ kpair[1] ^ np.uint32(0x1BD11BDA)))
    with np.errstate(over="ignore"):
        a = (x0 + ks[0]).astype(np.uint32)
        b = (x1 + ks[1]).astype(np.uint32)
        for i in range(5):
            for r in rot[i % 2]:
                a = (a + b).astype(np.uint32)
                b = rotl(b, r) ^ a
            a = (a + ks[(i + 1) % 3]).astype(np.uint32)
            b = (b + ks[(i + 2) % 3] + np.uint32(i + 1)).astype(np.uint32)
    return a, b


def _threefry_bits(kpair, size):
    a, b = _threefry2x32(kpair, np.zeros(size, np.uint32),
                         np.arange(size, dtype=np.uint32))
    return a ^ b


def _randint_key1(shape, span):
    a, b = _threefry2x32((np.uint32(0), np.uint32(1)),
                         np.zeros(2, np.uint32),
                         np.arange(2, dtype=np.uint32))
    key_hi = (a[0], b[0])
    key_lo = (a[1], b[1])
    n = int(np.prod(shape))
    hi = _threefry_bits(key_hi, n)
    lo = _threefry_bits(key_lo, n)
    s = np.uint32(span)
    with np.errstate(over="ignore"):
        mult = np.uint32(65536) % s
        mult = np.uint32(mult * mult) % s
        vals = (np.uint32(np.uint32(hi % s) * mult) + (lo % s)) % s
    return vals.astype(np.int64).reshape(shape)


def _flat_idx_base():
    idx = _randint_key1((BSZ, KP1), N_ROWS)
    b = np.arange(BSZ, dtype=np.int64)[None, :]
    flat = (idx.T * BSZ + b).reshape(-1)
    out = np.zeros((IDX_PAD,), dtype=np.int32)
    out[: flat.size] = flat.astype(np.int32)
    return out


_FLAT_IDX_BASE = _flat_idx_base()


def kernel(x, y, memory):
    p = _matmul(memory, x)
    p_flat = p.reshape(N_ROWS * BSZ)
    flat_idx = jnp.asarray(_FLAT_IDX_BASE)
    ycol = y.astype(jnp.int32) * BSZ + jnp.arange(BSZ, dtype=jnp.int32)
    out = _gather_kernel()(p_flat, flat_idx, ycol)
    logits = out.reshape(KP1, BSZ).T
    labels = jnp.zeros((BSZ,), dtype=jnp.int32)
    return (logits, labels)


# speedup vs baseline: 17.1144x; 1.0219x over previous
import functools

import numpy as np
import jax
import jax.numpy as jnp
from jax import lax
from jax.experimental import pallas as pl
from jax.experimental.pallas import tpu as pltpu
from jax.experimental.pallas import tpu_sc as plsc

N_ROWS = 100000
N_DIM = 256
KP1 = 2049
BSZ = 128
INV_T = 1.0 / 0.07

BM = 10000
GRID_M = N_ROWS // BM


def _matmul_body(mem_ref, x_ref, out_ref):
    xs = x_ref[...] * INV_T
    out_ref[...] = lax.dot_general(
        mem_ref[...], xs,
        dimension_numbers=(((1,), (1,)), ((), ())),
        preferred_element_type=jnp.float32,
        precision=lax.Precision.DEFAULT,
    )


def _matmul(memory, x):
    return pl.pallas_call(
        _matmul_body,
        grid=(GRID_M,),
        in_specs=[
            pl.BlockSpec((BM, N_DIM), lambda i: (i, 0)),
            pl.BlockSpec((BSZ, N_DIM), lambda i: (0, 0)),
        ],
        out_specs=pl.BlockSpec((BM, BSZ), lambda i: (i, 0)),
        out_shape=jax.ShapeDtypeStruct((N_ROWS, BSZ), jnp.float32),
    )(memory, x)


NW = 32
CHUNK = 128
W0_CHUNKS = 65
W_CHUNKS = 64
W0_N = W0_CHUNKS * CHUNK
W_N = W_CHUNKS * CHUNK
N_OUT = BSZ * KP1
IDX_PAD = W0_N + (NW - 1) * W_N + (W0_N - W_N)
DEPTH = 64


@functools.cache
def _gather_kernel():
    mesh = plsc.VectorSubcoreMesh(core_axis_name="c", subcore_axis_name="s")

    @functools.partial(
        pl.kernel,
        mesh=mesh,
        out_type=jax.ShapeDtypeStruct((N_OUT,), jnp.float32),
        scratch_types=[
            pltpu.VMEM((W0_N,), jnp.int32),
            pltpu.VMEM((W0_N,), jnp.float32),
            pltpu.SemaphoreType.DMA,
        ],
    )
    def gather(p_hbm, idx_hbm, ycol_hbm, out_hbm, idx_v, val_v, sem):
        wid = lax.axis_index("s") * 2 + lax.axis_index("c")
        base = jnp.where(wid == 0, 0, W0_N + (wid - 1) * W_N)
        nch = jnp.where(wid == 0, W0_CHUNKS, W_CHUNKS)
        pltpu.sync_copy(idx_hbm.at[pl.ds(pl.multiple_of(base, 8), W0_N)],
                        idx_v)

        @pl.when(wid == 0)
        def _():
            pltpu.sync_copy(ycol_hbm, idx_v.at[pl.ds(0, BSZ)])

        def issue(g):
            off = pl.multiple_of(g * CHUNK, CHUNK)
            return pltpu.async_copy(
                p_hbm.at[idx_v.at[pl.ds(off, CHUNK)]],
                val_v.at[pl.ds(off, CHUNK)], sem)

        def wait_one():
            pltpu.make_async_copy(p_hbm.at[pl.ds(0, CHUNK)],
                                  val_v.at[pl.ds(0, CHUNK)], sem).wait()

        def pro(g, carry):
            issue(g)
            return carry

        lax.fori_loop(0, DEPTH, pro, 0)

        def body(g, carry):
            issue(g + DEPTH)
            wait_one()
            return carry

        lax.fori_loop(0, nch - DEPTH, body, 0)

        def epi(g, carry):
            wait_one()
            return carry

        lax.fori_loop(0, DEPTH, epi, 0)

        @pl.when(wid == 0)
        def _():
            pltpu.sync_copy(val_v, out_hbm.at[pl.ds(0, W0_N)])

        @pl.when(wid > 0)
        def _():
            pltpu.sync_copy(
                val_v.at[pl.ds(0, W_N)],
                out_hbm.at[pl.ds(pl.multiple_of(base, 8), W_N)])

    return gather


def _threefry2x32(kpair, x0, x1):
    rot = ((13, 15, 26, 6), (17, 29, 16, 24))

    def rotl(v, d):
        return ((v << np.uint32(d)) | (v >> np.uint32(32 - d))).astype(np.uint32)

    ks = (np.uint32(kpair[0]), np.uint32(kpair[1]),
          np.uint32(kpair[0] ^ kpair[1] ^ np.uint32(0x1BD11BDA)))
    with np.errstate(over="ignore"):
        a = (x0 + ks[0]).astype(np.uint32)
        b = (x1 + ks[1]).astype(np.uint32)
        for i in range(5):
            for r in rot[i % 2]:
                a = (a + b).astype(np.uint32)
                b = rotl(b, r) ^ a
            a = (a + ks[(i + 1) % 3]).astype(np.uint32)
            b = (b + ks[(i + 2) % 3] + np.uint32(i + 1)).astype(np.uint32)
    return a, b


def _threefry_bits(kpair, size):
    a, b = _threefry2x32(kpair, np.zeros(size, np.uint32),
                         np.arange(size, dtype=np.uint32))
    return a ^ b


def _randint_key1(shape, span):
    a, b = _threefry2x32((np.uint32(0), np.uint32(1)),
                         np.zeros(2, np.uint32),
                         np.arange(2, dtype=np.uint32))
    key_hi = (a[0], b[0])
    key_lo = (a[1], b[1])
    n = int(np.prod(shape))
    hi = _threefry_bits(key_hi, n)
    lo = _threefry_bits(key_lo, n)
    s = np.uint32(span)
    with np.errstate(over="ignore"):
        mult = np.uint32(65536) % s
        mult = np.uint32(mult * mult) % s
        vals = (np.uint32(np.uint32(hi % s) * mult) + (lo % s)) % s
    return vals.astype(np.int64).reshape(shape)


def _flat_idx_base():
    idx = _randint_key1((BSZ, KP1), N_ROWS)
    b = np.arange(BSZ, dtype=np.int64)[None, :]
    flat = (idx.T * BSZ + b).reshape(-1)
    out = np.zeros((IDX_PAD,), dtype=np.int32)
    out[: flat.size] = flat.astype(np.int32)
    return out


_FLAT_IDX_BASE = _flat_idx_base()


def kernel(x, y, memory):
    p = _matmul(memory, x)
    p_flat = p.reshape(N_ROWS * BSZ)
    flat_idx = jnp.asarray(_FLAT_IDX_BASE)
    ycol = y.astype(jnp.int32) * BSZ + jnp.arange(BSZ, dtype=jnp.int32)
    out = _gather_kernel()(p_flat, flat_idx, ycol)
    logits = out.reshape(KP1, BSZ).T
    labels = jnp.zeros((BSZ,), dtype=jnp.int32)
    return (logits, labels)


# speedup vs baseline: 17.2852x; 1.0100x over previous
import functools

import numpy as np
import jax
import jax.numpy as jnp
from jax import lax
from jax.experimental import pallas as pl
from jax.experimental.pallas import tpu as pltpu
from jax.experimental.pallas import tpu_sc as plsc

N_ROWS = 100000
N_DIM = 256
KP1 = 2049
BSZ = 128
INV_T = 1.0 / 0.07

BM = 10000
GRID_M = N_ROWS // BM


def _matmul_body(mem_ref, x_ref, out_ref):
    xs = x_ref[...] * INV_T
    out_ref[...] = lax.dot_general(
        mem_ref[...], xs,
        dimension_numbers=(((1,), (1,)), ((), ())),
        preferred_element_type=jnp.float32,
        precision=lax.Precision.DEFAULT,
    )


def _matmul(memory, x):
    return pl.pallas_call(
        _matmul_body,
        grid=(GRID_M,),
        in_specs=[
            pl.BlockSpec((BM, N_DIM), lambda i: (i, 0)),
            pl.BlockSpec((BSZ, N_DIM), lambda i: (0, 0)),
        ],
        out_specs=pl.BlockSpec((BM, BSZ), lambda i: (i, 0)),
        out_shape=jax.ShapeDtypeStruct((N_ROWS, BSZ), jnp.float32),
    )(memory, x)


NW = 32
CHUNK = 128
W0_CHUNKS = 65
W_CHUNKS = 64
W0_N = W0_CHUNKS * CHUNK
W_N = W_CHUNKS * CHUNK
N_OUT = BSZ * KP1
IDX_PAD = W0_N + (NW - 1) * W_N + (W0_N - W_N)
DEPTH = 64


@functools.cache
def _gather_kernel():
    mesh = plsc.VectorSubcoreMesh(core_axis_name="c", subcore_axis_name="s")

    @functools.partial(
        pl.kernel,
        mesh=mesh,
        out_type=jax.ShapeDtypeStruct((N_OUT,), jnp.float32),
        scratch_types=[
            pltpu.VMEM((W0_N,), jnp.int32),
            pltpu.VMEM((W0_N,), jnp.float32),
            pltpu.VMEM((BSZ,), jnp.int32),
            pltpu.SemaphoreType.DMA,
        ],
    )
    def gather(p_hbm, idx_hbm, y_hbm, out_hbm, idx_v, val_v, y_v, sem):
        wid = lax.axis_index("s") * 2 + lax.axis_index("c")
        base = jnp.where(wid == 0, 0, W0_N + (wid - 1) * W_N)
        nch = jnp.where(wid == 0, W0_CHUNKS, W_CHUNKS)
        pltpu.sync_copy(idx_hbm.at[pl.ds(pl.multiple_of(base, 8), W0_N)],
                        idx_v)

        @pl.when(wid == 0)
        def _():
            pltpu.sync_copy(y_hbm, y_v)
            for c in range(BSZ // 16):
                yv = y_v[pl.ds(c * 16, 16)]
                lanes = lax.iota(jnp.int32, 16) + (c * 16)
                idx_v[pl.ds(c * 16, 16)] = yv * BSZ + lanes

        def issue(g):
            off = pl.multiple_of(g * CHUNK, CHUNK)
            return pltpu.async_copy(
                p_hbm.at[idx_v.at[pl.ds(off, CHUNK)]],
                val_v.at[pl.ds(off, CHUNK)], sem)

        def wait_one():
            pltpu.make_async_copy(p_hbm.at[pl.ds(0, CHUNK)],
                                  val_v.at[pl.ds(0, CHUNK)], sem).wait()

        def pro(g, carry):
            issue(g)
            return carry

        lax.fori_loop(0, DEPTH, pro, 0)

        def body(g, carry):
            issue(g + DEPTH)
            wait_one()
            return carry

        lax.fori_loop(0, nch - DEPTH, body, 0)

        def epi(g, carry):
            wait_one()
            return carry

        lax.fori_loop(0, DEPTH, epi, 0)

        @pl.when(wid == 0)
        def _():
            pltpu.sync_copy(val_v, out_hbm.at[pl.ds(0, W0_N)])

        @pl.when(wid > 0)
        def _():
            pltpu.sync_copy(
                val_v.at[pl.ds(0, W_N)],
                out_hbm.at[pl.ds(pl.multiple_of(base, 8), W_N)])

    return gather


def _threefry2x32(kpair, x0, x1):
    rot = ((13, 15, 26, 6), (17, 29, 16, 24))

    def rotl(v, d):
        return ((v << np.uint32(d)) | (v >> np.uint32(32 - d))).astype(np.uint32)

    ks = (np.uint32(kpair[0]), np.uint32(kpair[1]),
          np.uint32(kpair[0] ^ kpair[1] ^ np.uint32(0x1BD11BDA)))
    with np.errstate(over="ignore"):
        a = (x0 + ks[0]).astype(np.uint32)
        b = (x1 + ks[1]).astype(np.uint32)
        for i in range(5):
            for r in rot[i % 2]:
                a = (a + b).astype(np.uint32)
                b = rotl(b, r) ^ a
            a = (a + ks[(i + 1) % 3]).astype(np.uint32)
            b = (b + ks[(i + 2) % 3] + np.uint32(i + 1)).astype(np.uint32)
    return a, b


def _threefry_bits(kpair, size):
    a, b = _threefry2x32(kpair, np.zeros(size, np.uint32),
                         np.arange(size, dtype=np.uint32))
    return a ^ b


def _randint_key1(shape, span):
    a, b = _threefry2x32((np.uint32(0), np.uint32(1)),
                         np.zeros(2, np.uint32),
                         np.arange(2, dtype=np.uint32))
    key_hi = (a[0], b[0])
    key_lo = (a[1], b[1])
    n = int(np.prod(shape))
    hi = _threefry_bits(key_hi, n)
    lo = _threefry_bits(key_lo, n)
    s = np.uint32(span)
    with np.errstate(over="ignore"):
        mult = np.uint32(65536) % s
        mult = np.uint32(mult * mult) % s
        vals = (np.uint32(np.uint32(hi % s) * mult) + (lo % s)) % s
    return vals.astype(np.int64).reshape(shape)


def _flat_idx_base():
    idx = _randint_key1((BSZ, KP1), N_ROWS)
    b = np.arange(BSZ, dtype=np.int64)[None, :]
    flat = (idx.T * BSZ + b).reshape(-1)
    out = np.zeros((IDX_PAD,), dtype=np.int32)
    out[: flat.size] = flat.astype(np.int32)
    return out


_FLAT_IDX_BASE = _flat_idx_base()


def kernel(x, y, memory):
    p = _matmul(memory, x)
    p_flat = p.reshape(N_ROWS * BSZ)
    flat_idx = jnp.asarray(_FLAT_IDX_BASE)
    out = _gather_kernel()(p_flat, flat_idx, y.astype(jnp.int32))
    logits = out.reshape(KP1, BSZ).T
    labels = jnp.zeros((BSZ,), dtype=jnp.int32)
    return (logits, labels)


# speedup vs baseline: 17.5177x; 1.0135x over previous
import functools

import numpy as np
import jax
import jax.numpy as jnp
from jax import lax
from jax.experimental import pallas as pl
from jax.experimental.pallas import tpu as pltpu
from jax.experimental.pallas import tpu_sc as plsc

N_ROWS = 100000
N_DIM = 256
KP1 = 2049
BSZ = 128
INV_T = 1.0 / 0.07

BM = 12800
GRID_M = -(-N_ROWS // BM)


def _matmul_body(mem_ref, x_ref, out_ref):
    xs = x_ref[...] * INV_T
    out_ref[...] = lax.dot_general(
        mem_ref[...], xs,
        dimension_numbers=(((1,), (1,)), ((), ())),
        preferred_element_type=jnp.float32,
        precision=lax.Precision.DEFAULT,
    )


def _matmul(memory, x):
    return pl.pallas_call(
        _matmul_body,
        grid=(GRID_M,),
        in_specs=[
            pl.BlockSpec((BM, N_DIM), lambda i: (i, 0)),
            pl.BlockSpec((BSZ, N_DIM), lambda i: (0, 0)),
        ],
        out_specs=pl.BlockSpec((BM, BSZ), lambda i: (i, 0)),
        out_shape=jax.ShapeDtypeStruct((N_ROWS, BSZ), jnp.float32),
    )(memory, x)


NW = 32
CHUNK = 128
W0_CHUNKS = 65
W_CHUNKS = 64
W0_N = W0_CHUNKS * CHUNK
W_N = W_CHUNKS * CHUNK
N_OUT = BSZ * KP1
IDX_PAD = W0_N + (NW - 1) * W_N + (W0_N - W_N)
DEPTH = 64


@functools.cache
def _gather_kernel():
    mesh = plsc.VectorSubcoreMesh(core_axis_name="c", subcore_axis_name="s")

    @functools.partial(
        pl.kernel,
        mesh=mesh,
        out_type=jax.ShapeDtypeStruct((N_OUT,), jnp.float32),
        scratch_types=[
            pltpu.VMEM((W0_N,), jnp.int32),
            pltpu.VMEM((W0_N,), jnp.float32),
            pltpu.VMEM((BSZ,), jnp.int32),
            pltpu.SemaphoreType.DMA,
        ],
    )
    def gather(p_hbm, idx_hbm, y_hbm, out_hbm, idx_v, val_v, y_v, sem):
        wid = lax.axis_index("s") * 2 + lax.axis_index("c")
        base = jnp.where(wid == 0, 0, W0_N + (wid - 1) * W_N)
        nch = jnp.where(wid == 0, W0_CHUNKS, W_CHUNKS)
        pltpu.sync_copy(idx_hbm.at[pl.ds(pl.multiple_of(base, 8), W0_N)],
                        idx_v)

        @pl.when(wid == 0)
        def _():
            pltpu.sync_copy(y_hbm, y_v)
            for c in range(BSZ // 16):
                yv = y_v[pl.ds(c * 16, 16)]
                lanes = lax.iota(jnp.int32, 16) + (c * 16)
                idx_v[pl.ds(c * 16, 16)] = yv * BSZ + lanes

        def issue(g):
            off = pl.multiple_of(g * CHUNK, CHUNK)
            return pltpu.async_copy(
                p_hbm.at[idx_v.at[pl.ds(off, CHUNK)]],
                val_v.at[pl.ds(off, CHUNK)], sem)

        def wait_one():
            pltpu.make_async_copy(p_hbm.at[pl.ds(0, CHUNK)],
                                  val_v.at[pl.ds(0, CHUNK)], sem).wait()

        def pro(g, carry):
            issue(g)
            return carry

        lax.fori_loop(0, DEPTH, pro, 0)

        def body(g, carry):
            issue(g + DEPTH)
            wait_one()
            return carry

        lax.fori_loop(0, nch - DEPTH, body, 0)

        def epi(g, carry):
            wait_one()
            return carry

        lax.fori_loop(0, DEPTH, epi, 0)

        @pl.when(wid == 0)
        def _():
            pltpu.sync_copy(val_v, out_hbm.at[pl.ds(0, W0_N)])

        @pl.when(wid > 0)
        def _():
            pltpu.sync_copy(
                val_v.at[pl.ds(0, W_N)],
                out_hbm.at[pl.ds(pl.multiple_of(base, 8), W_N)])

    return gather


def _threefry2x32(kpair, x0, x1):
    rot = ((13, 15, 26, 6), (17, 29, 16, 24))

    def rotl(v, d):
        return ((v << np.uint32(d)) | (v >> np.uint32(32 - d))).astype(np.uint32)

    ks = (np.uint32(kpair[0]), np.uint32(kpair[1]),
          np.uint32(kpair[0] ^ kpair[1] ^ np.uint32(0x1BD11BDA)))
    with np.errstate(over="ignore"):
        a = (x0 + ks[0]).astype(np.uint32)
        b = (x1 + ks[1]).astype(np.uint32)
        for i in range(5):
            for r in rot[i % 2]:
                a = (a + b).astype(np.uint32)
                b = rotl(b, r) ^ a
            a = (a + ks[(i + 1) % 3]).astype(np.uint32)
            b = (b + ks[(i + 2) % 3] + np.uint32(i + 1)).astype(np.uint32)
    return a, b


def _threefry_bits(kpair, size):
    a, b = _threefry2x32(kpair, np.zeros(size, np.uint32),
                         np.arange(size, dtype=np.uint32))
    return a ^ b


def _randint_key1(shape, span):
    a, b = _threefry2x32((np.uint32(0), np.uint32(1)),
                         np.zeros(2, np.uint32),
                         np.arange(2, dtype=np.uint32))
    key_hi = (a[0], b[0])
    key_lo = (a[1], b[1])
    n = int(np.prod(shape))
    hi = _threefry_bits(key_hi, n)
    lo = _threefry_bits(key_lo, n)
    s = np.uint32(span)
    with np.errstate(over="ignore"):
        mult = np.uint32(65536) % s
        mult = np.uint32(mult * mult) % s
        vals = (np.uint32(np.uint32(hi % s) * mult) + (lo % s)) % s
    return vals.astype(np.int64).reshape(shape)


def _flat_idx_base():
    idx = _randint_key1((BSZ, KP1), N_ROWS)
    b = np.arange(BSZ, dtype=np.int64)[None, :]
    flat = (idx.T * BSZ + b).reshape(-1)
    out = np.zeros((IDX_PAD,), dtype=np.int32)
    out[: flat.size] = flat.astype(np.int32)
    return out


_FLAT_IDX_BASE = _flat_idx_base()


def kernel(x, y, memory):
    p = _matmul(memory, x)
    p_flat = p.reshape(N_ROWS * BSZ)
    flat_idx = jnp.asarray(_FLAT_IDX_BASE)
    out = _gather_kernel()(p_flat, flat_idx, y.astype(jnp.int32))
    logits = out.reshape(KP1, BSZ).T
    labels = jnp.zeros((BSZ,), dtype=jnp.int32)
    return (logits, labels)
